# baseline (speedup 1.0000x reference)
import functools
import jax
import jax.numpy as jnp
from jax.experimental import pallas as pl
from jax.experimental.pallas import tpu as pltpu

H = 1024
E = 8
FFH = H // E
LORA_R = 16
LORA_SCALE = 2.0
HID = E * FFH
LR = E * LORA_R
TB = 256


def _fused_kernel(x_ref, g_ref, win_ref, wout_ref, o_ref):
    xb = x_ref[...]

    logits = jnp.dot(xb, g_ref[...], preferred_element_type=jnp.float32)
    col = jax.lax.broadcasted_iota(jnp.int32, logits.shape, 1)
    logits = jnp.where(col < E, logits, -1e30)
    m1 = jnp.max(logits, axis=-1, keepdims=True)
    idx1 = jnp.min(jnp.where(logits == m1, col, E), axis=-1, keepdims=True)
    l2 = jnp.where(col == idx1, -1e30, logits)
    m2 = jnp.max(l2, axis=-1, keepdims=True)
    idx2 = jnp.min(jnp.where(l2 == m2, col, E), axis=-1, keepdims=True)
    t = jnp.exp(m2 - m1)
    w1 = 1.0 / (1.0 + t)
    w2 = t / (1.0 + t)

    xb16 = xb.astype(jnp.bfloat16)
    acts = jnp.dot(xb16, win_ref[...], preferred_element_type=jnp.float32)
    a = acts[:, :HID]
    u = acts[:, HID:2 * HID]
    l = acts[:, 2 * HID:]
    h = (a / (1.0 + jnp.exp(-a))) * u

    hcol = jax.lax.broadcasted_iota(jnp.int32, h.shape, 1) // FFH
    wh = jnp.where(hcol == idx1, w1, 0.0) + jnp.where(hcol == idx2, w2, 0.0)
    lcol = jax.lax.broadcasted_iota(jnp.int32, l.shape, 1) // LORA_R
    wl = jnp.where(lcol == idx1, w1, 0.0) + jnp.where(lcol == idx2, w2, 0.0)
    hw = jnp.concatenate(
        [(h * wh).astype(jnp.bfloat16), (l * (LORA_SCALE * wl)).astype(jnp.bfloat16)],
        axis=1)

    o_ref[...] = jnp.dot(hw, wout_ref[...], preferred_element_type=jnp.float32)


@functools.partial(jax.jit, static_argnames=("interpret",))
def _run(xt, g_pad, w_in, w_out, interpret=False):
    n = xt.shape[0]
    return pl.pallas_call(
        _fused_kernel,
        grid=(n // TB,),
        in_specs=[
            pl.BlockSpec((TB, H), lambda i: (i, 0)),
            pl.BlockSpec((H, 128), lambda i: (0, 0)),
            pl.BlockSpec((H, 2 * HID + LR), lambda i: (0, 0)),
            pl.BlockSpec((HID + LR, H), lambda i: (0, 0)),
        ],
        out_specs=pl.BlockSpec((TB, H), lambda i: (i, 0)),
        out_shape=jax.ShapeDtypeStruct((n, H), jnp.float32),
        compiler_params=pltpu.CompilerParams(
            dimension_semantics=("arbitrary",)),
        interpret=interpret,
    )(xt, g_pad, w_in, w_out)


def kernel(input, G, gate_proj, up_proj, down_proj, lora_A, lora_B,
           interpret=False):
    b, s, h = input.shape
    xt = input.reshape(-1, h)
    g_pad = jnp.pad(G, ((0, 0), (0, 128 - E)))
    gate_t = gate_proj.reshape(HID, H).T
    up_t = up_proj.reshape(HID, H).T
    la_t = lora_A.reshape(LR, H).T
    w_in = jnp.concatenate([gate_t, up_t, la_t], axis=1).astype(jnp.bfloat16)
    down_s = down_proj.transpose(0, 2, 1).reshape(HID, H)
    lb_s = lora_B.transpose(0, 2, 1).reshape(LR, H)
    w_out = jnp.concatenate([down_s, lb_s], axis=0).astype(jnp.bfloat16)
    out = _run(xt, g_pad, w_in, w_out, interpret=interpret)
    return out.reshape(b, s, h)


# speedup vs baseline: 1.0232x; 1.0232x over previous
import functools
import jax
import jax.numpy as jnp
from jax.experimental import pallas as pl
from jax.experimental.pallas import tpu as pltpu

H = 1024
E = 8
FFH = H // E
LORA_R = 16
LORA_SCALE = 2.0
HID = E * FFH
LR = E * LORA_R
TB = 1024


def _fused_kernel(x_ref, g_ref, win_ref, wout_ref, o_ref):
    xb = x_ref[...]

    logits = jnp.dot(xb, g_ref[...], preferred_element_type=jnp.float32)
    col = jax.lax.broadcasted_iota(jnp.int32, logits.shape, 1)
    logits = jnp.where(col < E, logits, -1e30)
    m1 = jnp.max(logits, axis=-1, keepdims=True)
    idx1 = jnp.min(jnp.where(logits == m1, col, E), axis=-1, keepdims=True)
    l2 = jnp.where(col == idx1, -1e30, logits)
    m2 = jnp.max(l2, axis=-1, keepdims=True)
    idx2 = jnp.min(jnp.where(l2 == m2, col, E), axis=-1, keepdims=True)
    t = jnp.exp(m2 - m1)
    w1 = 1.0 / (1.0 + t)
    w2 = t / (1.0 + t)

    xb16 = xb.astype(jnp.bfloat16)
    acts = jnp.dot(xb16, win_ref[...], preferred_element_type=jnp.float32)
    a = acts[:, :HID]
    u = acts[:, HID:2 * HID]
    l = acts[:, 2 * HID:]
    h = (a / (1.0 + jnp.exp(-a))) * u

    hcol = jax.lax.broadcasted_iota(jnp.int32, h.shape, 1) // FFH
    wh = jnp.where(hcol == idx1, w1, 0.0) + jnp.where(hcol == idx2, w2, 0.0)
    lcol = jax.lax.broadcasted_iota(jnp.int32, l.shape, 1) // LORA_R
    wl = jnp.where(lcol == idx1, w1, 0.0) + jnp.where(lcol == idx2, w2, 0.0)
    hw = jnp.concatenate(
        [(h * wh).astype(jnp.bfloat16), (l * (LORA_SCALE * wl)).astype(jnp.bfloat16)],
        axis=1)

    o_ref[...] = jnp.dot(hw, wout_ref[...], preferred_element_type=jnp.float32)


@functools.partial(jax.jit, static_argnames=("interpret",))
def _run(xt, g_pad, w_in, w_out, interpret=False):
    n = xt.shape[0]
    return pl.pallas_call(
        _fused_kernel,
        grid=(n // TB,),
        in_specs=[
            pl.BlockSpec((TB, H), lambda i: (i, 0)),
            pl.BlockSpec((H, 128), lambda i: (0, 0)),
            pl.BlockSpec((H, 2 * HID + LR), lambda i: (0, 0)),
            pl.BlockSpec((HID + LR, H), lambda i: (0, 0)),
        ],
        out_specs=pl.BlockSpec((TB, H), lambda i: (i, 0)),
        out_shape=jax.ShapeDtypeStruct((n, H), jnp.float32),
        compiler_params=pltpu.CompilerParams(
            dimension_semantics=("arbitrary",)),
        interpret=interpret,
    )(xt, g_pad, w_in, w_out)


def kernel(input, G, gate_proj, up_proj, down_proj, lora_A, lora_B,
           interpret=False):
    b, s, h = input.shape
    xt = input.reshape(-1, h)
    g_pad = jnp.pad(G, ((0, 0), (0, 128 - E)))
    gate_t = gate_proj.reshape(HID, H).T
    up_t = up_proj.reshape(HID, H).T
    la_t = lora_A.reshape(LR, H).T
    w_in = jnp.concatenate([gate_t, up_t, la_t], axis=1).astype(jnp.bfloat16)
    down_s = down_proj.transpose(0, 2, 1).reshape(HID, H)
    lb_s = lora_B.transpose(0, 2, 1).reshape(LR, H)
    w_out = jnp.concatenate([down_s, lb_s], axis=0).astype(jnp.bfloat16)
    out = _run(xt, g_pad, w_in, w_out, interpret=interpret)
    return out.reshape(b, s, h)


# speedup vs baseline: 1.2479x; 1.2196x over previous
import functools
import jax
import jax.numpy as jnp
from jax.experimental import pallas as pl
from jax.experimental.pallas import tpu as pltpu

H = 1024
E = 8
FFH = H // E
LORA_R = 16
LORA_SCALE = 2.0
HID = E * FFH
LR = E * LORA_R
TB = 512

_RHS_T = (((1,), (1,)), ((), ()))
_STD = (((1,), (0,)), ((), ()))


def _fused_kernel(x_ref, g_ref, gp_ref, up_ref, la_ref, dp_ref, lb_ref,
                  o_ref, win_s, wout_s):
    @pl.when(pl.program_id(0) == 0)
    def _prep():
        win_s[:HID, :] = gp_ref[...].astype(jnp.bfloat16)
        win_s[HID:2 * HID, :] = up_ref[...].astype(jnp.bfloat16)
        win_s[2 * HID:, :] = la_ref[...].astype(jnp.bfloat16)
        for e in range(E):
            wout_s[e * FFH:(e + 1) * FFH, :] = (
                dp_ref[e].T.astype(jnp.bfloat16))
            wout_s[HID + e * LORA_R:HID + (e + 1) * LORA_R, :] = (
                lb_ref[e].T.astype(jnp.bfloat16))

    xb = x_ref[...]

    logits = jnp.dot(xb, g_ref[...], preferred_element_type=jnp.float32)
    col = jax.lax.broadcasted_iota(jnp.int32, logits.shape, 1)
    logits = jnp.where(col < E, logits, -1e30)
    m1 = jnp.max(logits, axis=-1, keepdims=True)
    idx1 = jnp.min(jnp.where(logits == m1, col, E), axis=-1, keepdims=True)
    l2 = jnp.where(col == idx1, -1e30, logits)
    m2 = jnp.max(l2, axis=-1, keepdims=True)
    idx2 = jnp.min(jnp.where(l2 == m2, col, E), axis=-1, keepdims=True)
    t = jnp.exp(m2 - m1)
    w1 = 1.0 / (1.0 + t)
    w2 = t / (1.0 + t)

    xb16 = xb.astype(jnp.bfloat16)
    acts = jax.lax.dot_general(xb16, win_s[...], _RHS_T,
                               preferred_element_type=jnp.float32)
    a = acts[:, :HID]
    u = acts[:, HID:2 * HID]
    l = acts[:, 2 * HID:]
    h = (a / (1.0 + jnp.exp(-a))) * u

    hcol = jax.lax.broadcasted_iota(jnp.int32, h.shape, 1) // FFH
    wh = jnp.where(hcol == idx1, w1, 0.0) + jnp.where(hcol == idx2, w2, 0.0)
    lcol = jax.lax.broadcasted_iota(jnp.int32, l.shape, 1) // LORA_R
    wl = jnp.where(lcol == idx1, w1, 0.0) + jnp.where(lcol == idx2, w2, 0.0)
    hw = jnp.concatenate(
        [(h * wh).astype(jnp.bfloat16), (l * (LORA_SCALE * wl)).astype(jnp.bfloat16)],
        axis=1)

    o_ref[...] = jax.lax.dot_general(hw, wout_s[...], _STD,
                                     preferred_element_type=jnp.float32)


@functools.partial(jax.jit, static_argnames=("interpret",))
def _run(xt, g_pad, gp_r, up_r, la_r, dp, lb, interpret=False):
    n = xt.shape[0]
    full = lambda i: (0, 0)
    full3 = lambda i: (0, 0, 0)
    return pl.pallas_call(
        _fused_kernel,
        grid=(n // TB,),
        in_specs=[
            pl.BlockSpec((TB, H), lambda i: (i, 0)),
            pl.BlockSpec((H, 128), full),
            pl.BlockSpec((HID, H), full),
            pl.BlockSpec((HID, H), full),
            pl.BlockSpec((LR, H), full),
            pl.BlockSpec((E, H, FFH), full3),
            pl.BlockSpec((E, H, LORA_R), full3),
        ],
        out_specs=pl.BlockSpec((TB, H), lambda i: (i, 0)),
        out_shape=jax.ShapeDtypeStruct((n, H), jnp.float32),
        scratch_shapes=[
            pltpu.VMEM((2 * HID + LR, H), jnp.bfloat16),
            pltpu.VMEM((HID + LR, H), jnp.bfloat16),
        ],
        compiler_params=pltpu.CompilerParams(
            dimension_semantics=("arbitrary",)),
        interpret=interpret,
    )(xt, g_pad, gp_r, up_r, la_r, dp, lb)


def kernel(input, G, gate_proj, up_proj, down_proj, lora_A, lora_B,
           interpret=False):
    b, s, h = input.shape
    xt = input.reshape(-1, h)
    g_pad = jnp.pad(G, ((0, 0), (0, 128 - E)))
    gp_r = gate_proj.reshape(HID, H)
    up_r = up_proj.reshape(HID, H)
    la_r = lora_A.reshape(LR, H)
    out = _run(xt, g_pad, gp_r, up_r, la_r, down_proj, lora_B,
               interpret=interpret)
    return out.reshape(b, s, h)


# speedup vs baseline: 1.2591x; 1.0090x over previous
import functools
import jax
import jax.numpy as jnp
from jax.experimental import pallas as pl
from jax.experimental.pallas import tpu as pltpu

H = 1024
E = 8
FFH = H // E
LORA_R = 16
LORA_SCALE = 2.0
HID = E * FFH
LR = E * LORA_R
TB = 512

_RHS_T = (((1,), (1,)), ((), ()))
_STD = (((1,), (0,)), ((), ()))


def _fused_kernel(x_ref, g_ref, gp_ref, up_ref, la_ref, dp_ref, lb_ref,
                  o_ref, win_s, wout_s):
    @pl.when(pl.program_id(0) == 0)
    def _prep():
        win_s[:, :HID] = gp_ref[...].T.astype(jnp.bfloat16)
        win_s[:, HID:2 * HID] = up_ref[...].T.astype(jnp.bfloat16)
        win_s[:, 2 * HID:] = la_ref[...].T.astype(jnp.bfloat16)
        for e in range(E):
            wout_s[e * FFH:(e + 1) * FFH, :] = (
                dp_ref[e].T.astype(jnp.bfloat16))
            wout_s[HID + e * LORA_R:HID + (e + 1) * LORA_R, :] = (
                lb_ref[e].T.astype(jnp.bfloat16))

    xb = x_ref[...]

    logits = jnp.dot(xb, g_ref[...], preferred_element_type=jnp.float32)
    col = jax.lax.broadcasted_iota(jnp.int32, logits.shape, 1)
    logits = jnp.where(col < E, logits, -1e30)
    m1 = jnp.max(logits, axis=-1, keepdims=True)
    idx1 = jnp.min(jnp.where(logits == m1, col, E), axis=-1, keepdims=True)
    l2 = jnp.where(col == idx1, -1e30, logits)
    m2 = jnp.max(l2, axis=-1, keepdims=True)
    idx2 = jnp.min(jnp.where(l2 == m2, col, E), axis=-1, keepdims=True)
    t = jnp.exp(m2 - m1)
    w1 = 1.0 / (1.0 + t)
    w2 = t / (1.0 + t)

    xb16 = xb.astype(jnp.bfloat16)
    acts = jax.lax.dot_general(xb16, win_s[...], _STD,
                               preferred_element_type=jnp.float32)
    a = acts[:, :HID]
    u = acts[:, HID:2 * HID]
    l = acts[:, 2 * HID:]
    h = (a / (1.0 + jnp.exp(-a))) * u

    hcol = jax.lax.broadcasted_iota(jnp.int32, h.shape, 1) // FFH
    wh = jnp.where(hcol == idx1, w1, 0.0) + jnp.where(hcol == idx2, w2, 0.0)
    lcol = jax.lax.broadcasted_iota(jnp.int32, l.shape, 1) // LORA_R
    wl = jnp.where(lcol == idx1, w1, 0.0) + jnp.where(lcol == idx2, w2, 0.0)
    hw = jnp.concatenate(
        [(h * wh).astype(jnp.bfloat16), (l * (LORA_SCALE * wl)).astype(jnp.bfloat16)],
        axis=1)

    o_ref[...] = jax.lax.dot_general(hw, wout_s[...], _STD,
                                     preferred_element_type=jnp.float32)


@functools.partial(jax.jit, static_argnames=("interpret",))
def _run(xt, g_pad, gp_r, up_r, la_r, dp, lb, interpret=False):
    n = xt.shape[0]
    full = lambda i: (0, 0)
    full3 = lambda i: (0, 0, 0)
    return pl.pallas_call(
        _fused_kernel,
        grid=(n // TB,),
        in_specs=[
            pl.BlockSpec((TB, H), lambda i: (i, 0)),
            pl.BlockSpec((H, 128), full),
            pl.BlockSpec((HID, H), full),
            pl.BlockSpec((HID, H), full),
            pl.BlockSpec((LR, H), full),
            pl.BlockSpec((E, H, FFH), full3),
            pl.BlockSpec((E, H, LORA_R), full3),
        ],
        out_specs=pl.BlockSpec((TB, H), lambda i: (i, 0)),
        out_shape=jax.ShapeDtypeStruct((n, H), jnp.float32),
        scratch_shapes=[
            pltpu.VMEM((H, 2 * HID + LR), jnp.bfloat16),
            pltpu.VMEM((HID + LR, H), jnp.bfloat16),
        ],
        compiler_params=pltpu.CompilerParams(
            dimension_semantics=("arbitrary",)),
        interpret=interpret,
    )(xt, g_pad, gp_r, up_r, la_r, dp, lb)


def kernel(input, G, gate_proj, up_proj, down_proj, lora_A, lora_B,
           interpret=False):
    b, s, h = input.shape
    xt = input.reshape(-1, h)
    g_pad = jnp.pad(G, ((0, 0), (0, 128 - E)))
    gp_r = gate_proj.reshape(HID, H)
    up_r = up_proj.reshape(HID, H)
    la_r = lora_A.reshape(LR, H)
    out = _run(xt, g_pad, gp_r, up_r, la_r, down_proj, lora_B,
               interpret=interpret)
    return out.reshape(b, s, h)


# speedup vs baseline: 1.2746x; 1.0123x over previous
import functools
import jax
import jax.numpy as jnp
from jax.experimental import pallas as pl
from jax.experimental.pallas import tpu as pltpu

H = 1024
E = 8
FFH = H // E
LORA_R = 16
LORA_SCALE = 2.0
HID = E * FFH
LR = E * LORA_R
TB = 512


def _fused_kernel(x_ref, g_ref, gp_hbm, up_hbm, la_hbm, dp_hbm, lb_hbm,
                  o_ref, win_s, wout_s, sa, sb, sc, sd, se, sems):
    @pl.when(pl.program_id(0) == 0)
    def _prep():
        cpa = pltpu.make_async_copy(gp_hbm, sa, sems.at[0])
        cpb = pltpu.make_async_copy(up_hbm, sb, sems.at[1])
        cpc = pltpu.make_async_copy(la_hbm, sc, sems.at[2])
        cpd = pltpu.make_async_copy(dp_hbm, sd, sems.at[3])
        cpe = pltpu.make_async_copy(lb_hbm, se, sems.at[4])
        cpa.start(); cpb.start(); cpc.start(); cpd.start(); cpe.start()
        cpa.wait()
        win_s[:, :HID] = sa[...].T.astype(jnp.bfloat16)
        cpb.wait()
        win_s[:, HID:2 * HID] = sb[...].T.astype(jnp.bfloat16)
        cpc.wait()
        win_s[:, 2 * HID:] = sc[...].T.astype(jnp.bfloat16)
        cpd.wait()
        for e in range(E):
            wout_s[e * FFH:(e + 1) * FFH, :] = sd[e].T.astype(jnp.bfloat16)
        cpe.wait()
        for e in range(E):
            wout_s[HID + e * LORA_R:HID + (e + 1) * LORA_R, :] = (
                se[e].T.astype(jnp.bfloat16))

    xb = x_ref[...]

    logits = jnp.dot(xb, g_ref[...], preferred_element_type=jnp.float32)
    col = jax.lax.broadcasted_iota(jnp.int32, logits.shape, 1)
    logits = jnp.where(col < E, logits, -1e30)
    m1 = jnp.max(logits, axis=-1, keepdims=True)
    idx1 = jnp.min(jnp.where(logits == m1, col, E), axis=-1, keepdims=True)
    l2 = jnp.where(col == idx1, -1e30, logits)
    m2 = jnp.max(l2, axis=-1, keepdims=True)
    idx2 = jnp.min(jnp.where(l2 == m2, col, E), axis=-1, keepdims=True)
    t = jnp.exp(m2 - m1)
    w1 = 1.0 / (1.0 + t)
    w2 = t / (1.0 + t)

    xb16 = xb.astype(jnp.bfloat16)
    a = jnp.dot(xb16, win_s[:, :HID], preferred_element_type=jnp.float32)
    u = jnp.dot(xb16, win_s[:, HID:2 * HID], preferred_element_type=jnp.float32)
    l = jnp.dot(xb16, win_s[:, 2 * HID:], preferred_element_type=jnp.float32)
    h = (a / (1.0 + jnp.exp(-a))) * u

    hcol = jax.lax.broadcasted_iota(jnp.int32, h.shape, 1) // FFH
    wh = jnp.where(hcol == idx1, w1, 0.0) + jnp.where(hcol == idx2, w2, 0.0)
    lcol = jax.lax.broadcasted_iota(jnp.int32, l.shape, 1) // LORA_R
    wl = jnp.where(lcol == idx1, w1, 0.0) + jnp.where(lcol == idx2, w2, 0.0)
    hw = (h * wh).astype(jnp.bfloat16)
    lw = (l * (LORA_SCALE * wl)).astype(jnp.bfloat16)

    o_ref[...] = (
        jnp.dot(hw, wout_s[:HID, :], preferred_element_type=jnp.float32)
        + jnp.dot(lw, wout_s[HID:, :], preferred_element_type=jnp.float32))


@functools.partial(jax.jit, static_argnames=("interpret",))
def _run(xt, g_pad, gp_r, up_r, la_r, dp, lb, interpret=False):
    n = xt.shape[0]
    anyspec = pl.BlockSpec(memory_space=pl.ANY)
    return pl.pallas_call(
        _fused_kernel,
        grid=(n // TB,),
        in_specs=[
            pl.BlockSpec((TB, H), lambda i: (i, 0)),
            pl.BlockSpec((H, 128), lambda i: (0, 0)),
            anyspec, anyspec, anyspec, anyspec, anyspec,
        ],
        out_specs=pl.BlockSpec((TB, H), lambda i: (i, 0)),
        out_shape=jax.ShapeDtypeStruct((n, H), jnp.float32),
        scratch_shapes=[
            pltpu.VMEM((H, 2 * HID + LR), jnp.bfloat16),
            pltpu.VMEM((HID + LR, H), jnp.bfloat16),
            pltpu.VMEM((HID, H), jnp.float32),
            pltpu.VMEM((HID, H), jnp.float32),
            pltpu.VMEM((LR, H), jnp.float32),
            pltpu.VMEM((E, H, FFH), jnp.float32),
            pltpu.VMEM((E, H, LORA_R), jnp.float32),
            pltpu.SemaphoreType.DMA((5,)),
        ],
        compiler_params=pltpu.CompilerParams(
            dimension_semantics=("arbitrary",)),
        interpret=interpret,
    )(xt, g_pad, gp_r, up_r, la_r, dp, lb)


def kernel(input, G, gate_proj, up_proj, down_proj, lora_A, lora_B,
           interpret=False):
    b, s, h = input.shape
    xt = input.reshape(-1, h)
    g_pad = jnp.pad(G, ((0, 0), (0, 128 - E)))
    gp_r = gate_proj.reshape(HID, H)
    up_r = up_proj.reshape(HID, H)
    la_r = lora_A.reshape(LR, H)
    out = _run(xt, g_pad, gp_r, up_r, la_r, down_proj, lora_B,
               interpret=interpret)
    return out.reshape(b, s, h)
